# baseline (speedup 1.0000x reference)
import functools

import jax
import jax.numpy as jnp
from jax import lax
from jax.experimental import pallas as pl
from jax.experimental.pallas import tpu as pltpu
from jax.experimental.pallas import tpu_sc as plsc

D_MODEL = 1024
NUM_SEG = 10
NC, NS, L = 2, 16, 16
NW = NC * NS
BLK = 512
SC_UNROLL = 4


def _make_onehot_sc(T):
    tpw = T // NW
    mesh = plsc.VectorSubcoreMesh(core_axis_name="c", subcore_axis_name="s")

    @functools.partial(
        pl.kernel,
        out_type=jax.ShapeDtypeStruct((T, L), jnp.float32),
        mesh=mesh,
        compiler_params=pltpu.CompilerParams(
            use_tc_tiling_on_sc=False, needs_layout_passes=False
        ),
        scratch_types=[
            pltpu.VMEM((tpw,), jnp.int32),
            pltpu.VMEM((tpw, L), jnp.float32),
        ],
    )
    def body(ids_hbm, oh_hbm, ids_v, oh_v):
        wid = lax.axis_index("s") * NC + lax.axis_index("c")
        base = wid * tpw
        pltpu.sync_copy(ids_hbm.at[pl.ds(base, tpw)], ids_v)
        iota = lax.iota(jnp.int32, L)
        one = jnp.ones((L,), jnp.float32)
        zero = jnp.zeros((L,), jnp.float32)

        @plsc.parallel_loop(0, tpw, unroll=SC_UNROLL)
        def tok_body(t):
            r_vec = plsc.load_gather(ids_v, [jnp.broadcast_to(t, (L,))])
            oh_v[t, :] = jnp.where(iota == r_vec, one, zero)

        pltpu.sync_copy(oh_v, oh_hbm.at[pl.ds(base, tpw)])

    return body


def _tc_call(x2, oh2, table16, T):
    nblk = T // BLK

    def body(oh_ref, x_ref, tab_ref, o_ref):
        seg = jnp.dot(
            oh_ref[...], tab_ref[...], preferred_element_type=jnp.float32
        )
        o_ref[...] = x_ref[...] + seg

    return pl.pallas_call(
        body,
        grid=(nblk,),
        in_specs=[
            pl.BlockSpec((BLK, L), lambda i: (i, 0)),
            pl.BlockSpec((BLK, D_MODEL), lambda i: (i, 0)),
            pl.BlockSpec((L, D_MODEL), lambda i: (0, 0)),
        ],
        out_specs=pl.BlockSpec((BLK, D_MODEL), lambda i: (i, 0)),
        out_shape=jax.ShapeDtypeStruct((T, D_MODEL), jnp.float32),
    )(oh2, x2, table16)


def kernel(x, segment_ids, table):
    B, S, D = x.shape
    T = B * S
    ids = segment_ids.reshape(T).astype(jnp.int32)
    table16 = jnp.concatenate(
        [table, jnp.zeros((L - NUM_SEG, D), table.dtype)], axis=0
    )
    oh = _make_onehot_sc(T)(ids)
    out = _tc_call(x.reshape(T, D), oh, table16, T)
    return out.reshape(B, S, D)


# speedup vs baseline: 1.0745x; 1.0745x over previous
import functools

import jax
import jax.numpy as jnp
from jax import lax
from jax.experimental import pallas as pl
from jax.experimental.pallas import tpu as pltpu
from jax.experimental.pallas import tpu_sc as plsc

D_MODEL = 1024
NUM_SEG = 10
NC, NS, L = 2, 16, 16
NW = NC * NS
BLK = 1024
SC_UNROLL = 4


def _make_onehot_sc(T):
    tpw = T // NW
    mesh = plsc.VectorSubcoreMesh(core_axis_name="c", subcore_axis_name="s")

    @functools.partial(
        pl.kernel,
        out_type=jax.ShapeDtypeStruct((T, L), jnp.float32),
        mesh=mesh,
        compiler_params=pltpu.CompilerParams(
            use_tc_tiling_on_sc=False, needs_layout_passes=False
        ),
        scratch_types=[
            pltpu.VMEM((tpw,), jnp.int32),
            pltpu.VMEM((tpw, L), jnp.float32),
        ],
    )
    def body(ids_hbm, oh_hbm, ids_v, oh_v):
        wid = lax.axis_index("s") * NC + lax.axis_index("c")
        base = wid * tpw
        pltpu.sync_copy(ids_hbm.at[pl.ds(base, tpw)], ids_v)
        iota = lax.iota(jnp.int32, L)
        one = jnp.ones((L,), jnp.float32)
        zero = jnp.zeros((L,), jnp.float32)

        @plsc.parallel_loop(0, tpw, unroll=SC_UNROLL)
        def tok_body(t):
            r_vec = plsc.load_gather(ids_v, [jnp.broadcast_to(t, (L,))])
            oh_v[t, :] = jnp.where(iota == r_vec, one, zero)

        pltpu.sync_copy(oh_v, oh_hbm.at[pl.ds(base, tpw)])

    return body


def _tc_call(x2, oh2, table16, T):
    nblk = T // BLK

    def body(oh_ref, x_ref, tab_ref, o_ref):
        seg = jnp.dot(
            oh_ref[...], tab_ref[...], preferred_element_type=jnp.float32
        )
        o_ref[...] = x_ref[...] + seg

    return pl.pallas_call(
        body,
        grid=(nblk,),
        in_specs=[
            pl.BlockSpec((BLK, L), lambda i: (i, 0)),
            pl.BlockSpec((BLK, D_MODEL), lambda i: (i, 0)),
            pl.BlockSpec((L, D_MODEL), lambda i: (0, 0)),
        ],
        out_specs=pl.BlockSpec((BLK, D_MODEL), lambda i: (i, 0)),
        out_shape=jax.ShapeDtypeStruct((T, D_MODEL), jnp.float32),
    )(oh2, x2, table16)


def kernel(x, segment_ids, table):
    B, S, D = x.shape
    T = B * S
    ids = segment_ids.reshape(T).astype(jnp.int32)
    table16 = jnp.concatenate(
        [table, jnp.zeros((L - NUM_SEG, D), table.dtype)], axis=0
    )
    oh = _make_onehot_sc(T)(ids)
    out = _tc_call(x.reshape(T, D), oh, table16, T)
    return out.reshape(B, S, D)


# speedup vs baseline: 1.0857x; 1.0104x over previous
import functools

import jax
import jax.numpy as jnp
from jax import lax
from jax.experimental import pallas as pl
from jax.experimental.pallas import tpu as pltpu
from jax.experimental.pallas import tpu_sc as plsc

D_MODEL = 1024
NUM_SEG = 10
NC, NS, L = 2, 16, 16
NW = NC * NS
BLK = 2048
SC_UNROLL = 4


def _make_onehot_sc(T):
    tpw = T // NW
    mesh = plsc.VectorSubcoreMesh(core_axis_name="c", subcore_axis_name="s")

    @functools.partial(
        pl.kernel,
        out_type=jax.ShapeDtypeStruct((T, L), jnp.float32),
        mesh=mesh,
        compiler_params=pltpu.CompilerParams(
            use_tc_tiling_on_sc=False, needs_layout_passes=False
        ),
        scratch_types=[
            pltpu.VMEM((tpw,), jnp.int32),
            pltpu.VMEM((tpw, L), jnp.float32),
        ],
    )
    def body(ids_hbm, oh_hbm, ids_v, oh_v):
        wid = lax.axis_index("s") * NC + lax.axis_index("c")
        base = wid * tpw
        pltpu.sync_copy(ids_hbm.at[pl.ds(base, tpw)], ids_v)
        iota = lax.iota(jnp.int32, L)
        one = jnp.ones((L,), jnp.float32)
        zero = jnp.zeros((L,), jnp.float32)

        @plsc.parallel_loop(0, tpw, unroll=SC_UNROLL)
        def tok_body(t):
            r_vec = plsc.load_gather(ids_v, [jnp.broadcast_to(t, (L,))])
            oh_v[t, :] = jnp.where(iota == r_vec, one, zero)

        pltpu.sync_copy(oh_v, oh_hbm.at[pl.ds(base, tpw)])

    return body


def _tc_call(x2, oh2, table16, T):
    nblk = T // BLK

    def body(oh_ref, x_ref, tab_ref, o_ref):
        seg = jnp.dot(
            oh_ref[...], tab_ref[...], preferred_element_type=jnp.float32
        )
        o_ref[...] = x_ref[...] + seg

    return pl.pallas_call(
        body,
        grid=(nblk,),
        in_specs=[
            pl.BlockSpec((BLK, L), lambda i: (i, 0)),
            pl.BlockSpec((BLK, D_MODEL), lambda i: (i, 0)),
            pl.BlockSpec((L, D_MODEL), lambda i: (0, 0)),
        ],
        out_specs=pl.BlockSpec((BLK, D_MODEL), lambda i: (i, 0)),
        out_shape=jax.ShapeDtypeStruct((T, D_MODEL), jnp.float32),
    )(oh2, x2, table16)


def kernel(x, segment_ids, table):
    B, S, D = x.shape
    T = B * S
    ids = segment_ids.reshape(T).astype(jnp.int32)
    table16 = jnp.concatenate(
        [table, jnp.zeros((L - NUM_SEG, D), table.dtype)], axis=0
    )
    oh = _make_onehot_sc(T)(ids)
    out = _tc_call(x.reshape(T, D), oh, table16, T)
    return out.reshape(B, S, D)
